# baseline (speedup 1.0000x reference)
import jax
import jax.numpy as jnp
from jax import lax
from jax.experimental import pallas as pl
from jax.experimental.pallas import tpu as pltpu

K = 10
N_ENT = 100000
DIM = 64
B = 1024

BS_T = 256
ENT_T = 512
LANES = 128
SUBS = ENT_T // LANES
ENT_PAD = 100352
NJ = ENT_PAD // ENT_T
NEG = -3.0e38
BIG = 1 << 30


def _topk_body(hr_ref, e_ref, out_v_ref, out_i_ref, best_v, best_i, cand_v, cand_i):
    j = pl.program_id(1)

    @pl.when(j == 0)
    def _init():
        for t in range(K):
            best_v[t] = jnp.full((BS_T, LANES), NEG, jnp.float32)
            best_i[t] = jnp.zeros((BS_T, LANES), jnp.int32)

    hr = hr_ref[...]
    e = e_ref[...]
    scores = lax.dot_general(
        hr, e, (((1,), (1,)), ((), ())),
        preferred_element_type=jnp.float32,
    )

    lane_iota = lax.broadcasted_iota(jnp.int32, (BS_T, LANES), 1)
    for sub in range(SUBS):
        v = scores[:, sub * LANES:(sub + 1) * LANES]
        col0 = j * ENT_T + sub * LANES
        vi = lane_iota + col0
        v = jnp.where(vi < N_ENT, v, NEG)
        for t in range(K):
            bv = best_v[t]
            bi = best_i[t]
            gt = v > bv
            best_v[t] = jnp.where(gt, v, bv)
            best_i[t] = jnp.where(gt, vi, bi)
            v = jnp.where(gt, bv, v)
            vi = jnp.where(gt, bi, vi)

    @pl.when(j == NJ - 1)
    def _extract():
        for t in range(K):
            cand_v[:, t * LANES:(t + 1) * LANES] = best_v[t]
            cand_i[:, t * LANES:(t + 1) * LANES] = best_i[t]
        iota = lax.broadcasted_iota(jnp.int32, (BS_T, K * LANES), 1)
        for t in range(K):
            cv = cand_v[...]
            m = jnp.max(cv, axis=1, keepdims=True)
            hit = cv == m
            pos = jnp.min(jnp.where(hit, iota, BIG), axis=1, keepdims=True)
            sel = iota == pos
            win_i = jnp.sum(jnp.where(sel, cand_i[...], 0), axis=1, keepdims=True)
            out_v_ref[:, pl.ds(t, 1)] = m
            out_i_ref[:, pl.ds(t, 1)] = win_i
            cand_v[...] = jnp.where(sel, NEG, cv)


def kernel(relation, head, entity_embedding, relation_embedding):
    rel = relation.reshape(-1)
    hd = head.reshape(-1)
    head_emb = jnp.take(entity_embedding, hd, axis=0)
    rel_emb = jnp.take(relation_embedding, rel, axis=0)
    hr = head_emb * rel_emb

    e_pad = jnp.pad(entity_embedding, ((0, ENT_PAD - N_ENT), (0, 0)))

    out_v, out_i = pl.pallas_call(
        _topk_body,
        grid=(B // BS_T, NJ),
        in_specs=[
            pl.BlockSpec((BS_T, DIM), lambda i, j: (i, 0)),
            pl.BlockSpec((ENT_T, DIM), lambda i, j: (j, 0)),
        ],
        out_specs=[
            pl.BlockSpec((BS_T, LANES), lambda i, j: (i, 0)),
            pl.BlockSpec((BS_T, LANES), lambda i, j: (i, 0)),
        ],
        out_shape=[
            jax.ShapeDtypeStruct((B, LANES), jnp.float32),
            jax.ShapeDtypeStruct((B, LANES), jnp.int32),
        ],
        scratch_shapes=[
            pltpu.VMEM((K, BS_T, LANES), jnp.float32),
            pltpu.VMEM((K, BS_T, LANES), jnp.int32),
            pltpu.VMEM((BS_T, K * LANES), jnp.float32),
            pltpu.VMEM((BS_T, K * LANES), jnp.int32),
        ],
    )(hr, e_pad)

    return out_v[:, :K], out_i[:, :K]
